# baseline (speedup 1.0000x reference)
import functools
import math

import jax
import jax.numpy as jnp
from jax import lax
from jax.experimental import pallas as pl
from jax.experimental.pallas import tpu as pltpu
from jax.experimental.pallas import tpu_sc as plsc

TOP_K = 8
NEG = -1e30


def _nt_dot(a, b):
    return lax.dot_general(a, b, (((1,), (1,)), ((), ())),
                           preferred_element_type=jnp.float32)


def _proj_body(x_ref, wq_ref, bq_ref, wk_ref, bk_ref, q_ref, k_ref):
    q_ref[0] = _nt_dot(x_ref[0], wq_ref[...]) + bq_ref[...]
    k_ref[0] = _nt_dot(x_ref[0], wk_ref[...]) + bk_ref[...]


def _project(action_states, Wq, bq, Wk, bk):
    B, N, D = action_states.shape
    DQ = Wq.shape[0]
    return pl.pallas_call(
        _proj_body,
        grid=(B,),
        in_specs=[
            pl.BlockSpec((1, N, D), lambda b: (b, 0, 0)),
            pl.BlockSpec((DQ, D), lambda b: (0, 0)),
            pl.BlockSpec((1, DQ), lambda b: (0, 0)),
            pl.BlockSpec((DQ, D), lambda b: (0, 0)),
            pl.BlockSpec((1, DQ), lambda b: (0, 0)),
        ],
        out_specs=[
            pl.BlockSpec((1, N, DQ), lambda b: (b, 0, 0)),
            pl.BlockSpec((1, N, DQ), lambda b: (b, 0, 0)),
        ],
        out_shape=[
            jax.ShapeDtypeStruct((B, N, DQ), jnp.float32),
            jax.ShapeDtypeStruct((B, N, DQ), jnp.float32),
        ],
    )(action_states, Wq, bq.reshape(1, DQ), Wk, bk.reshape(1, DQ))


def _tc_body(q_ref, k_ref, vals_ref, idx_ref, *, scale):
    s = _nt_dot(q_ref[0], k_ref[0]) * (1.0 / scale)

    n = s.shape[1]
    iota = lax.broadcasted_iota(jnp.int32, s.shape, 1)
    w = s
    vals_l, idx_l = [], []
    for _ in range(TOP_K):
        mk = jnp.max(w, axis=1, keepdims=True)
        eq = w == mk
        ik = jnp.min(jnp.where(eq, iota, n), axis=1, keepdims=True)
        w = jnp.where(eq, NEG, w)
        vals_l.append(mk)
        idx_l.append(ik)
    vals8 = jnp.concatenate(vals_l, axis=1)
    idx8 = jnp.concatenate(idx_l, axis=1)
    e = jnp.exp(vals8 - vals8[:, :1])
    vals_ref[0] = e / jnp.sum(e, axis=1, keepdims=True)
    idx_ref[0] = idx8


def _tc_topk_compact(Q, K, scale, blk_r=256):
    B, N, DQ = Q.shape
    nb = N // blk_r
    body = functools.partial(_tc_body, scale=scale)
    vals, idx = pl.pallas_call(
        body,
        grid=(B, nb),
        in_specs=[
            pl.BlockSpec((1, blk_r, DQ), lambda b, j: (b, j, 0)),
            pl.BlockSpec((1, N, DQ), lambda b, j: (b, 0, 0)),
        ],
        out_specs=[
            pl.BlockSpec((1, blk_r, TOP_K), lambda b, j: (b, j, 0)),
            pl.BlockSpec((1, blk_r, TOP_K), lambda b, j: (b, j, 0)),
        ],
        out_shape=[
            jax.ShapeDtypeStruct((B, N, TOP_K), jnp.float32),
            jax.ShapeDtypeStruct((B, N, TOP_K), jnp.int32),
        ],
    )(Q, K)
    return vals, idx


def _sc_scatter(vals_flat, idx_flat, B, N):
    NC, NS = 2, 16
    NW = NC * NS
    n_rows = B * N
    n_pairs = n_rows // 2
    pairs_per_w = n_pairs // NW
    mesh = plsc.VectorSubcoreMesh(core_axis_name="c", subcore_axis_name="s",
                                  num_cores=NC, num_subcores=NS)

    @functools.partial(
        pl.kernel, mesh=mesh,
        compiler_params=pltpu.CompilerParams(needs_layout_passes=False),
        out_type=jax.ShapeDtypeStruct((n_rows * N,), jnp.float32),
        scratch_types=[
            pltpu.VMEM((pairs_per_w * 16,), jnp.int32),
            pltpu.VMEM((pairs_per_w * 16,), jnp.float32),
            pltpu.VMEM((2 * N,), jnp.float32),
            pltpu.VMEM((2 * N,), jnp.float32),
            pltpu.SemaphoreType.DMA,
            pltpu.SemaphoreType.DMA,
        ],
    )
    def k(vals_hbm, idx_hbm, out_hbm, idx_v, vals_v, buf0, buf1, sem0, sem1):
        wid = lax.axis_index("s") * NC + lax.axis_index("c")
        base = wid * pairs_per_w * 16
        pltpu.sync_copy(idx_hbm.at[pl.ds(base, pairs_per_w * 16)], idx_v)
        pltpu.sync_copy(vals_hbm.at[pl.ds(base, pairs_per_w * 16)], vals_v)

        zero = jnp.zeros((16,), jnp.float32)

        def zbody(i, c):
            buf0[pl.ds(i * 16, 16)] = zero
            buf1[pl.ds(i * 16, 16)] = zero
            return c

        lax.fori_loop(0, (2 * N) // 16, zbody, 0)

        lane = lax.iota(jnp.int32, 16)
        row_off = jnp.where(lane >= 8, N, 0)
        pair0 = wid * pairs_per_w

        def pbody(i, c):
            iv = idx_v[pl.ds(i * 16, 16)] + row_off
            vv = vals_v[pl.ds(i * 16, 16)]

            def run(buf, sem):
                @pl.when(i >= 2)
                def _():
                    pltpu.make_async_copy(
                        buf, out_hbm.at[pl.ds(0, 2 * N)], sem).wait()
                    pv = idx_v[pl.ds((i - 2) * 16, 16)] + row_off
                    plsc.store_scatter(buf, [pv], zero)

                plsc.store_scatter(buf, [iv], vv)
                dst = out_hbm.at[pl.ds((pair0 + i) * 2 * N, 2 * N)]
                pltpu.async_copy(buf, dst, sem)

            @pl.when(i % 2 == 0)
            def _():
                run(buf0, sem0)

            @pl.when(i % 2 == 1)
            def _():
                run(buf1, sem1)

            return c

        lax.fori_loop(0, pairs_per_w, pbody, 0)

        pltpu.make_async_copy(buf0, out_hbm.at[pl.ds(0, 2 * N)], sem0).wait()
        pltpu.make_async_copy(buf1, out_hbm.at[pl.ds(0, 2 * N)], sem1).wait()

    return k(vals_flat, idx_flat)


def kernel(action_states, Wq, bq, Wk, bk):
    B, N, _ = action_states.shape
    DQ = Wq.shape[0]
    Q, K = _project(action_states, Wq, bq, Wk, bk)
    vals, idx = _tc_topk_compact(Q, K, math.sqrt(DQ))
    out = _sc_scatter(vals.reshape(B * N * TOP_K), idx.reshape(B * N * TOP_K),
                      B, N)
    return out.reshape(B, N, N)


# speedup vs baseline: 1.3523x; 1.3523x over previous
import functools
import math

import jax
import jax.numpy as jnp
from jax import lax
from jax.experimental import pallas as pl
from jax.experimental.pallas import tpu as pltpu
from jax.experimental.pallas import tpu_sc as plsc

TOP_K = 8
NEG = -1e30


def _nt_dot(a, b):
    return lax.dot_general(a, b, (((1,), (1,)), ((), ())),
                           preferred_element_type=jnp.float32)


def _proj_body(x_ref, wq_ref, bq_ref, wk_ref, bk_ref, q_ref, k_ref):
    q_ref[0] = _nt_dot(x_ref[0], wq_ref[...]) + bq_ref[...]
    k_ref[0] = _nt_dot(x_ref[0], wk_ref[...]) + bk_ref[...]


def _project(action_states, Wq, bq, Wk, bk):
    B, N, D = action_states.shape
    DQ = Wq.shape[0]
    return pl.pallas_call(
        _proj_body,
        grid=(B,),
        in_specs=[
            pl.BlockSpec((1, N, D), lambda b: (b, 0, 0)),
            pl.BlockSpec((DQ, D), lambda b: (0, 0)),
            pl.BlockSpec((1, DQ), lambda b: (0, 0)),
            pl.BlockSpec((DQ, D), lambda b: (0, 0)),
            pl.BlockSpec((1, DQ), lambda b: (0, 0)),
        ],
        out_specs=[
            pl.BlockSpec((1, N, DQ), lambda b: (b, 0, 0)),
            pl.BlockSpec((1, N, DQ), lambda b: (b, 0, 0)),
        ],
        out_shape=[
            jax.ShapeDtypeStruct((B, N, DQ), jnp.float32),
            jax.ShapeDtypeStruct((B, N, DQ), jnp.float32),
        ],
    )(action_states, Wq, bq.reshape(1, DQ), Wk, bk.reshape(1, DQ))


def _tc_body(q_ref, k_ref, vals_ref, idx_ref, *, scale):
    s = _nt_dot(q_ref[0], k_ref[0]) * (1.0 / scale)

    n = s.shape[1]
    iota = lax.broadcasted_iota(jnp.int32, s.shape, 1)
    w = s
    vals_l, idx_l = [], []
    for _ in range(TOP_K):
        mk = jnp.max(w, axis=1, keepdims=True)
        eq = w == mk
        ik = jnp.min(jnp.where(eq, iota, n), axis=1, keepdims=True)
        w = jnp.where(eq, NEG, w)
        vals_l.append(mk)
        idx_l.append(ik)
    vals8 = jnp.concatenate(vals_l, axis=1)
    idx8 = jnp.concatenate(idx_l, axis=1)
    e = jnp.exp(vals8 - vals8[:, :1])
    vals_ref[0] = e / jnp.sum(e, axis=1, keepdims=True)
    idx_ref[0] = idx8


def _tc_topk_compact(Q, K, scale, blk_r=256):
    B, N, DQ = Q.shape
    nb = N // blk_r
    body = functools.partial(_tc_body, scale=scale)
    vals, idx = pl.pallas_call(
        body,
        grid=(B, nb),
        in_specs=[
            pl.BlockSpec((1, blk_r, DQ), lambda b, j: (b, j, 0)),
            pl.BlockSpec((1, N, DQ), lambda b, j: (b, 0, 0)),
        ],
        out_specs=[
            pl.BlockSpec((1, blk_r, TOP_K), lambda b, j: (b, j, 0)),
            pl.BlockSpec((1, blk_r, TOP_K), lambda b, j: (b, j, 0)),
        ],
        out_shape=[
            jax.ShapeDtypeStruct((B, N, TOP_K), jnp.float32),
            jax.ShapeDtypeStruct((B, N, TOP_K), jnp.int32),
        ],
    )(Q, K)
    return vals, idx


def _sc_scatter(vals_flat, idx_flat, B, N):
    NC, NS = 2, 16
    NW = NC * NS
    n_rows = B * N
    octs_per_w = n_rows // 8 // NW
    mesh = plsc.VectorSubcoreMesh(core_axis_name="c", subcore_axis_name="s",
                                  num_cores=NC, num_subcores=NS)

    @functools.partial(
        pl.kernel, mesh=mesh,
        compiler_params=pltpu.CompilerParams(needs_layout_passes=False,
                                             use_tc_tiling_on_sc=True),
        out_type=jax.ShapeDtypeStruct((n_rows, N), jnp.float32),
        scratch_types=[
            pltpu.VMEM((octs_per_w * 64,), jnp.int32),
            pltpu.VMEM((octs_per_w * 64,), jnp.float32),
            pltpu.VMEM((8, N), jnp.float32),
            pltpu.VMEM((8, N), jnp.float32),
            pltpu.SemaphoreType.DMA,
            pltpu.SemaphoreType.DMA,
        ],
    )
    def k(vals_hbm, idx_hbm, out_hbm, idx_v, vals_v, buf0, buf1, sem0, sem1):
        wid = lax.axis_index("s") * NC + lax.axis_index("c")
        base = wid * octs_per_w * 64
        pltpu.sync_copy(idx_hbm.at[pl.ds(base, octs_per_w * 64)], idx_v)
        pltpu.sync_copy(vals_hbm.at[pl.ds(base, octs_per_w * 64)], vals_v)

        zero = jnp.zeros((16,), jnp.float32)
        zrow = jnp.zeros((16,), jnp.int32)

        def zbody(i, c):
            buf0[i % 8, pl.ds((i // 8) * 16, 16)] = zero
            buf1[i % 8, pl.ds((i // 8) * 16, 16)] = zero
            return c

        lax.fori_loop(0, 8 * (N // 16), zbody, 0)

        lane = lax.iota(jnp.int32, 16)
        rowpair = lane // 8
        oct0 = wid * octs_per_w

        def pbody(i, c):
            def run(buf, sem):
                @pl.when(i >= 2)
                def _():
                    pltpu.make_async_copy(
                        buf, out_hbm.at[pl.ds(0, 8), :], sem).wait()
                    for v in range(4):
                        pv = idx_v[pl.ds((i - 2) * 64 + v * 16, 16)]
                        pr = rowpair + 2 * v
                        plsc.store_scatter(buf, [pr, pv], zero)

                for v in range(4):
                    iv = idx_v[pl.ds(i * 64 + v * 16, 16)]
                    vv = vals_v[pl.ds(i * 64 + v * 16, 16)]
                    pr = rowpair + 2 * v
                    plsc.store_scatter(buf, [pr, iv], vv)
                dst = out_hbm.at[pl.ds((oct0 + i) * 8, 8), :]
                pltpu.async_copy(buf, dst, sem)

            @pl.when(i % 2 == 0)
            def _():
                run(buf0, sem0)

            @pl.when(i % 2 == 1)
            def _():
                run(buf1, sem1)

            return c

        lax.fori_loop(0, octs_per_w, pbody, 0)

        pltpu.make_async_copy(buf0, out_hbm.at[pl.ds(0, 8), :], sem0).wait()
        pltpu.make_async_copy(buf1, out_hbm.at[pl.ds(0, 8), :], sem1).wait()

    return k(vals_flat, idx_flat)


def kernel(action_states, Wq, bq, Wk, bk):
    B, N, _ = action_states.shape
    DQ = Wq.shape[0]
    Q, K = _project(action_states, Wq, bq, Wk, bk)
    vals, idx = _tc_topk_compact(Q, K, math.sqrt(DQ))
    out = _sc_scatter(vals.reshape(B * N * TOP_K), idx.reshape(B * N * TOP_K),
                      B, N)
    return out.reshape(B, N, N)


# speedup vs baseline: 1.3872x; 1.0258x over previous
import functools
import math

import jax
import jax.numpy as jnp
from jax import lax
from jax.experimental import pallas as pl
from jax.experimental.pallas import tpu as pltpu
from jax.experimental.pallas import tpu_sc as plsc

TOP_K = 8
NEG = -1e30


def _nt_dot(a, b):
    return lax.dot_general(a, b, (((1,), (1,)), ((), ())),
                           preferred_element_type=jnp.float32)


def _proj_body(x_ref, wq_ref, bq_ref, wk_ref, bk_ref, q_ref, k_ref):
    q_ref[0] = _nt_dot(x_ref[0], wq_ref[...]) + bq_ref[...]
    k_ref[0] = _nt_dot(x_ref[0], wk_ref[...]) + bk_ref[...]


def _project(action_states, Wq, bq, Wk, bk):
    B, N, D = action_states.shape
    DQ = Wq.shape[0]
    return pl.pallas_call(
        _proj_body,
        grid=(B,),
        in_specs=[
            pl.BlockSpec((1, N, D), lambda b: (b, 0, 0)),
            pl.BlockSpec((DQ, D), lambda b: (0, 0)),
            pl.BlockSpec((1, DQ), lambda b: (0, 0)),
            pl.BlockSpec((DQ, D), lambda b: (0, 0)),
            pl.BlockSpec((1, DQ), lambda b: (0, 0)),
        ],
        out_specs=[
            pl.BlockSpec((1, N, DQ), lambda b: (b, 0, 0)),
            pl.BlockSpec((1, N, DQ), lambda b: (b, 0, 0)),
        ],
        out_shape=[
            jax.ShapeDtypeStruct((B, N, DQ), jnp.float32),
            jax.ShapeDtypeStruct((B, N, DQ), jnp.float32),
        ],
    )(action_states, Wq, bq.reshape(1, DQ), Wk, bk.reshape(1, DQ))


def _tc_body(q_ref, k_ref, vals_ref, idx_ref, *, scale):
    s = _nt_dot(q_ref[0], k_ref[0]) * (1.0 / scale)

    n = s.shape[1]
    iota = lax.broadcasted_iota(jnp.int32, s.shape, 1)
    w = s
    vals_l, idx_l = [], []
    for _ in range(TOP_K):
        mk = jnp.max(w, axis=1, keepdims=True)
        eq = w == mk
        ik = jnp.min(jnp.where(eq, iota, n), axis=1, keepdims=True)
        w = jnp.where(eq, NEG, w)
        vals_l.append(mk)
        idx_l.append(ik)
    vals8 = jnp.concatenate(vals_l, axis=1)
    idx8 = jnp.concatenate(idx_l, axis=1)
    e = jnp.exp(vals8 - vals8[:, :1])
    vals_ref[0] = e / jnp.sum(e, axis=1, keepdims=True)
    idx_ref[0] = idx8


def _tc_topk_compact(Q, K, scale, blk_r=512):
    B, N, DQ = Q.shape
    nb = N // blk_r
    body = functools.partial(_tc_body, scale=scale)
    vals, idx = pl.pallas_call(
        body,
        grid=(B, nb),
        in_specs=[
            pl.BlockSpec((1, blk_r, DQ), lambda b, j: (b, j, 0)),
            pl.BlockSpec((1, N, DQ), lambda b, j: (b, 0, 0)),
        ],
        out_specs=[
            pl.BlockSpec((1, blk_r, TOP_K), lambda b, j: (b, j, 0)),
            pl.BlockSpec((1, blk_r, TOP_K), lambda b, j: (b, j, 0)),
        ],
        out_shape=[
            jax.ShapeDtypeStruct((B, N, TOP_K), jnp.float32),
            jax.ShapeDtypeStruct((B, N, TOP_K), jnp.int32),
        ],
    )(Q, K)
    return vals, idx


def _sc_scatter(vals_flat, idx_flat, B, N):
    NC, NS = 2, 16
    NW = NC * NS
    n_rows = B * N
    octs_per_w = n_rows // 8 // NW
    mesh = plsc.VectorSubcoreMesh(core_axis_name="c", subcore_axis_name="s",
                                  num_cores=NC, num_subcores=NS)

    @functools.partial(
        pl.kernel, mesh=mesh,
        compiler_params=pltpu.CompilerParams(needs_layout_passes=False,
                                             use_tc_tiling_on_sc=True),
        out_type=jax.ShapeDtypeStruct((n_rows, N), jnp.float32),
        scratch_types=[
            pltpu.VMEM((octs_per_w * 64,), jnp.int32),
            pltpu.VMEM((octs_per_w * 64,), jnp.float32),
            pltpu.VMEM((8, N), jnp.float32),
            pltpu.VMEM((8, N), jnp.float32),
            pltpu.SemaphoreType.DMA,
            pltpu.SemaphoreType.DMA,
        ],
    )
    def k(vals_hbm, idx_hbm, out_hbm, idx_v, vals_v, buf0, buf1, sem0, sem1):
        wid = lax.axis_index("s") * NC + lax.axis_index("c")
        base = wid * octs_per_w * 64
        pltpu.sync_copy(idx_hbm.at[pl.ds(base, octs_per_w * 64)], idx_v)
        pltpu.sync_copy(vals_hbm.at[pl.ds(base, octs_per_w * 64)], vals_v)

        zero = jnp.zeros((16,), jnp.float32)
        zrow = jnp.zeros((16,), jnp.int32)

        def zbody(i, c):
            buf0[i % 8, pl.ds((i // 8) * 16, 16)] = zero
            buf1[i % 8, pl.ds((i // 8) * 16, 16)] = zero
            return c

        lax.fori_loop(0, 8 * (N // 16), zbody, 0)

        lane = lax.iota(jnp.int32, 16)
        rowpair = lane // 8
        oct0 = wid * octs_per_w

        def pbody(i, c):
            def run(buf, sem):
                @pl.when(i >= 2)
                def _():
                    pltpu.make_async_copy(
                        buf, out_hbm.at[pl.ds(0, 8), :], sem).wait()
                    for v in range(4):
                        pv = idx_v[pl.ds((i - 2) * 64 + v * 16, 16)]
                        pr = rowpair + 2 * v
                        plsc.store_scatter(buf, [pr, pv], zero)

                for v in range(4):
                    iv = idx_v[pl.ds(i * 64 + v * 16, 16)]
                    vv = vals_v[pl.ds(i * 64 + v * 16, 16)]
                    pr = rowpair + 2 * v
                    plsc.store_scatter(buf, [pr, iv], vv)
                dst = out_hbm.at[pl.ds((oct0 + i) * 8, 8), :]
                pltpu.async_copy(buf, dst, sem)

            @pl.when(i % 2 == 0)
            def _():
                run(buf0, sem0)

            @pl.when(i % 2 == 1)
            def _():
                run(buf1, sem1)

            return c

        lax.fori_loop(0, octs_per_w, pbody, 0)

        pltpu.make_async_copy(buf0, out_hbm.at[pl.ds(0, 8), :], sem0).wait()
        pltpu.make_async_copy(buf1, out_hbm.at[pl.ds(0, 8), :], sem1).wait()

    return k(vals_flat, idx_flat)


def kernel(action_states, Wq, bq, Wk, bk):
    B, N, _ = action_states.shape
    DQ = Wq.shape[0]
    Q, K = _project(action_states, Wq, bq, Wk, bk)
    vals, idx = _tc_topk_compact(Q, K, math.sqrt(DQ))
    out = _sc_scatter(vals.reshape(B * N * TOP_K), idx.reshape(B * N * TOP_K),
                      B, N)
    return out.reshape(B, N, N)
